# Initial kernel scaffold; baseline (speedup 1.0000x reference)
import jax
import jax.numpy as jnp
from jax.experimental import pallas as pl


def kernel(u, sample_index, amat_sample, offsets):
    raise NotImplementedError("write your pallas kernel here")


# speedup vs baseline: 6.3974x; 6.3974x over previous
import functools

import jax
import jax.numpy as jnp
from jax import lax
from jax.experimental import pallas as pl
from jax.experimental.pallas import tpu as pltpu
from jax.experimental.pallas import tpu_sc as plsc

N_SAMPLE = 100000
N_LATENT = 16
N_OUT = 16
BATCH = 16384

NC = 2
NS = 16
NW = NC * NS
ROWS_PER_W = BATCH // NW
CHUNK = 128
N_CHUNKS = ROWS_PER_W // CHUNK


def _sc_body(u_hbm, idx_hbm, amat_hbm, offs_hbm, out_hbm,
             idx_v, a_v, off_v, u_v, out_v, sem_a, sem_o):
    wid = lax.axis_index("s") * NC + lax.axis_index("c")
    base = wid * ROWS_PER_W

    lane_ids = [jnp.full((16,), l, dtype=jnp.int32) for l in range(N_LATENT)]

    for c in range(N_CHUNKS):
        cbase = base + c * CHUNK
        pltpu.sync_copy(idx_hbm.at[pl.ds(cbase, CHUNK)], idx_v)
        cp_a = pltpu.async_copy(amat_hbm.at[idx_v], a_v, sem_a)
        cp_o = pltpu.async_copy(offs_hbm.at[idx_v], off_v, sem_o)
        pltpu.sync_copy(u_hbm.at[pl.ds(cbase, CHUNK)], u_v)
        cp_a.wait()
        cp_o.wait()

        def row_body(r, carry):
            uvec = u_v[r, :]
            acc = uvec + off_v[r, :]
            for l in range(N_LATENT):
                a_l = a_v[r, pl.ds(l * 16, 16)]
                u_l = uvec.at[lane_ids[l]].get(mode="promise_in_bounds")
                acc = acc + u_l * a_l
            out_v[r, :] = acc
            return carry

        lax.fori_loop(0, CHUNK, row_body, 0)
        pltpu.sync_copy(out_v, out_hbm.at[pl.ds(cbase, CHUNK)])


@jax.jit
def kernel(u, sample_index, amat_sample, offsets):
    idx = jnp.squeeze(sample_index).astype(jnp.int32)
    amat2d = amat_sample.reshape(N_SAMPLE, N_LATENT * N_OUT)

    mesh = plsc.VectorSubcoreMesh(
        core_axis_name="c", subcore_axis_name="s",
        num_cores=NC, num_subcores=NS)
    run = pl.kernel(
        _sc_body,
        out_type=jax.ShapeDtypeStruct((BATCH, N_OUT), jnp.float32),
        mesh=mesh,
        scratch_types=[
            pltpu.VMEM((CHUNK,), jnp.int32),
            pltpu.VMEM((CHUNK, N_LATENT * N_OUT), jnp.float32),
            pltpu.VMEM((CHUNK, N_OUT), jnp.float32),
            pltpu.VMEM((CHUNK, N_LATENT), jnp.float32),
            pltpu.VMEM((CHUNK, N_OUT), jnp.float32),
            pltpu.SemaphoreType.DMA,
            pltpu.SemaphoreType.DMA,
        ],
        compiler_params=pltpu.CompilerParams(use_tc_tiling_on_sc=False),
    )
    return run(u, idx, amat2d, offsets)
